# Initial kernel scaffold; baseline (speedup 1.0000x reference)
import jax
import jax.numpy as jnp
from jax.experimental import pallas as pl


def kernel(x, edge_index, perm):
    raise NotImplementedError("write your pallas kernel here")


# speedup vs baseline: 111.9745x; 111.9745x over previous
import functools

import jax
import jax.numpy as jnp
from jax import lax
from jax.experimental import pallas as pl
from jax.experimental.pallas import tpu as pltpu
from jax.experimental.pallas import tpu_sc as plsc

N_NODES = 10000
D_FEAT = 128
N_EDGES = 320000

NC = 2
NS = 16
W = NC * NS
L = 16

N_PAD = 10240
ROWS_PW = N_PAD // W
RCH = 64
NRCH = ROWS_PW // RCH
EC = 2 * N_EDGES // W
SCAT_IT = N_PAD // L
GATH_IT = EC // L


def _sc_body(x_hbm, edge_hbm, pg_hbm, ps_hbm, out_x, out_e,
             perm_v, table_v, eidx_v, eout_v, xidx_v, rows_v, sem):
    cid = lax.axis_index("c")
    sid = lax.axis_index("s")
    wid = sid * NC + cid

    pltpu.sync_copy(ps_hbm, perm_v)

    def sbody(i, carry):
        base = pl.multiple_of(i * L, L)
        idxs = perm_v[pl.ds(base, L)]
        vals = lax.iota(jnp.int32, L) + i * L
        plsc.store_scatter(table_v, [idxs], vals)
        return carry

    lax.fori_loop(0, SCAT_IT, sbody, 0)

    pltpu.sync_copy(edge_hbm.at[wid], eidx_v)

    def gbody(i, carry):
        base = pl.multiple_of(i * L, L)
        idxs = eidx_v[pl.ds(base, L)]
        eout_v[pl.ds(base, L)] = plsc.load_gather(table_v, [idxs])
        return carry

    lax.fori_loop(0, GATH_IT, gbody, 0)
    pltpu.sync_copy(eout_v, out_e.at[wid])

    pltpu.sync_copy(pg_hbm.at[wid], xidx_v)
    copies = [
        pltpu.async_copy(x_hbm.at[xidx_v.at[j]], rows_v.at[j], sem)
        for j in range(NRCH)
    ]
    for j in range(NRCH):
        copies[j].wait()
        pltpu.sync_copy(
            rows_v.at[j],
            out_x.at[pl.ds(wid * ROWS_PW + j * RCH, RCH)])


_sc_call = functools.partial(
    pl.kernel,
    out_type=[
        jax.ShapeDtypeStruct((N_PAD, D_FEAT), jnp.float32),
        jax.ShapeDtypeStruct((W, EC), jnp.int32),
    ],
    mesh=plsc.VectorSubcoreMesh(core_axis_name="c", subcore_axis_name="s"),
    compiler_params=pltpu.CompilerParams(needs_layout_passes=False),
    scratch_types=[
        pltpu.VMEM((N_PAD,), jnp.int32),
        pltpu.VMEM((N_PAD,), jnp.int32),
        pltpu.VMEM((EC,), jnp.int32),
        pltpu.VMEM((EC,), jnp.int32),
        pltpu.VMEM((NRCH, RCH), jnp.int32),
        pltpu.VMEM((NRCH, RCH, D_FEAT), jnp.float32),
        pltpu.SemaphoreType.DMA,
    ],
)(_sc_body)


def kernel(x, edge_index, perm):
    perm32 = perm.astype(jnp.int32)
    e32 = edge_index.astype(jnp.int32)
    pg = jnp.concatenate(
        [perm32, jnp.zeros((N_PAD - N_NODES,), jnp.int32)]
    ).reshape(W, NRCH, RCH)
    ps = jnp.concatenate(
        [perm32, jnp.arange(N_NODES, N_PAD, dtype=jnp.int32)])
    ef = e32.reshape(W, EC)
    out_x, out_e = _sc_call(x, ef, pg, ps)
    return out_x[:N_NODES], out_e.reshape(2, N_EDGES).astype(edge_index.dtype)


# speedup vs baseline: 141.8677x; 1.2670x over previous
import functools

import jax
import jax.numpy as jnp
from jax import lax
from jax.experimental import pallas as pl
from jax.experimental.pallas import tpu as pltpu
from jax.experimental.pallas import tpu_sc as plsc

N_NODES = 10000
D_FEAT = 128
N_EDGES = 320000

NC = 2
NS = 16
W = NC * NS
L = 16

N_PAD = 10240
ROWS_PW = N_PAD // W
RCH = 64
NRCH = ROWS_PW // RCH
EC = 2 * N_EDGES // W
SCAT_IT = N_PAD // L
GATH_IT = EC // L


def _sc_body(x_hbm, edge_hbm, pg_hbm, ps_hbm, out_x, out_e,
             perm_v, table_v, eidx_v, eout_v, xidx_v, rows_v,
             sem_p, sem_e, sem_g):
    cid = lax.axis_index("c")
    sid = lax.axis_index("s")
    wid = sid * NC + cid

    perm_cp = pltpu.async_copy(ps_hbm, perm_v, sem_p)
    edge_cp = pltpu.async_copy(edge_hbm.at[wid], eidx_v, sem_e)
    pltpu.sync_copy(pg_hbm.at[wid], xidx_v)
    row_cps = [
        pltpu.async_copy(x_hbm.at[xidx_v.at[j]], rows_v.at[j], sem_g)
        for j in range(NRCH)
    ]

    perm_cp.wait()

    @plsc.parallel_loop(0, SCAT_IT * L, step=L, unroll=8)
    def _scat(base):
        base = pl.multiple_of(base, L)
        idxs = perm_v[pl.ds(base, L)]
        vals = lax.iota(jnp.int32, L) + base
        plsc.store_scatter(table_v, [idxs], vals)

    edge_cp.wait()

    @plsc.parallel_loop(0, GATH_IT * L, step=L, unroll=8)
    def _gath(base):
        base = pl.multiple_of(base, L)
        idxs = eidx_v[pl.ds(base, L)]
        eout_v[pl.ds(base, L)] = plsc.load_gather(table_v, [idxs])

    pltpu.sync_copy(eout_v, out_e.at[wid])

    for j in range(NRCH):
        row_cps[j].wait()
        pltpu.sync_copy(
            rows_v.at[j],
            out_x.at[pl.ds(wid * ROWS_PW + j * RCH, RCH)])


_sc_call = functools.partial(
    pl.kernel,
    out_type=[
        jax.ShapeDtypeStruct((N_PAD, D_FEAT), jnp.float32),
        jax.ShapeDtypeStruct((W, EC), jnp.int32),
    ],
    mesh=plsc.VectorSubcoreMesh(core_axis_name="c", subcore_axis_name="s"),
    compiler_params=pltpu.CompilerParams(needs_layout_passes=False),
    scratch_types=[
        pltpu.VMEM((N_PAD,), jnp.int32),
        pltpu.VMEM((N_PAD,), jnp.int32),
        pltpu.VMEM((EC,), jnp.int32),
        pltpu.VMEM((EC,), jnp.int32),
        pltpu.VMEM((NRCH, RCH), jnp.int32),
        pltpu.VMEM((NRCH, RCH, D_FEAT), jnp.float32),
        pltpu.SemaphoreType.DMA,
        pltpu.SemaphoreType.DMA,
        pltpu.SemaphoreType.DMA,
    ],
)(_sc_body)


def kernel(x, edge_index, perm):
    perm32 = perm.astype(jnp.int32)
    e32 = edge_index.astype(jnp.int32)
    pg = jnp.concatenate(
        [perm32, jnp.zeros((N_PAD - N_NODES,), jnp.int32)]
    ).reshape(W, NRCH, RCH)
    ps = jnp.concatenate(
        [perm32, jnp.arange(N_NODES, N_PAD, dtype=jnp.int32)])
    ef = e32.reshape(W, EC)
    out_x, out_e = _sc_call(x, ef, pg, ps)
    return out_x[:N_NODES], out_e.reshape(2, N_EDGES).astype(edge_index.dtype)


# speedup vs baseline: 204.9588x; 1.4447x over previous
import functools

import jax
import jax.numpy as jnp
from jax import lax
from jax.experimental import pallas as pl
from jax.experimental.pallas import tpu as pltpu
from jax.experimental.pallas import tpu_sc as plsc

N_NODES = 10000
D_FEAT = 128
N_EDGES = 320000

NC = 2
NS = 16
W = NC * NS
L = 16

ROWS_PW = 320
NRCH = ROWS_PW // L
EC = 2 * N_EDGES // W
SCAT_IT = N_NODES // L
GATH_IT = EC // L


def _sc_body(x_hbm, edge_hbm, perm_hbm, out_x, out_e,
             perm_v, table_v, eidx_v, eout_v, rows_v,
             sem_p, sem_e, sem_g):
    cid = lax.axis_index("c")
    sid = lax.axis_index("s")
    wid = sid * NC + cid
    ec = pl.multiple_of(wid * EC, EC)

    perm_cp = pltpu.async_copy(perm_hbm, perm_v, sem_p)
    edge_cp = pltpu.async_copy(edge_hbm.at[pl.ds(ec, EC)], eidx_v, sem_e)
    perm_cp.wait()
    row_cps = []
    for j in range(NRCH):
        base = jnp.minimum(wid * ROWS_PW + j * L, N_NODES - L)
        base = pl.multiple_of(base, 8)
        row_cps.append(pltpu.async_copy(
            x_hbm.at[perm_v.at[pl.ds(base, L)]], rows_v.at[j], sem_g))

    @plsc.parallel_loop(0, SCAT_IT * L, step=L, unroll=8)
    def _scat(sbase):
        sbase = pl.multiple_of(sbase, L)
        idxs = perm_v[pl.ds(sbase, L)]
        vals = lax.iota(jnp.int32, L) + sbase
        plsc.store_scatter(table_v, [idxs], vals)

    edge_cp.wait()

    @plsc.parallel_loop(0, GATH_IT * L, step=L, unroll=8)
    def _gath(gbase):
        gbase = pl.multiple_of(gbase, L)
        idxs = eidx_v[pl.ds(gbase, L)]
        eout_v[pl.ds(gbase, L)] = plsc.load_gather(table_v, [idxs])

    pltpu.sync_copy(eout_v, out_e.at[pl.ds(ec, EC)])

    for j in range(NRCH):
        row_cps[j].wait()
        base = pl.multiple_of(wid * ROWS_PW + j * L, L)

        @pl.when(base < N_NODES)
        def _():
            pltpu.sync_copy(rows_v.at[j], out_x.at[pl.ds(base, L)])


_sc_call = functools.partial(
    pl.kernel,
    out_type=[
        jax.ShapeDtypeStruct((N_NODES, D_FEAT), jnp.float32),
        jax.ShapeDtypeStruct((2 * N_EDGES,), jnp.int32),
    ],
    mesh=plsc.VectorSubcoreMesh(core_axis_name="c", subcore_axis_name="s"),
    compiler_params=pltpu.CompilerParams(needs_layout_passes=False),
    scratch_types=[
        pltpu.VMEM((N_NODES,), jnp.int32),
        pltpu.VMEM((N_NODES,), jnp.int32),
        pltpu.VMEM((EC,), jnp.int32),
        pltpu.VMEM((EC,), jnp.int32),
        pltpu.VMEM((NRCH, L, D_FEAT), jnp.float32),
        pltpu.SemaphoreType.DMA,
        pltpu.SemaphoreType.DMA,
        pltpu.SemaphoreType.DMA,
    ],
)(_sc_body)


def kernel(x, edge_index, perm):
    out_x, out_e = _sc_call(
        x, edge_index.astype(jnp.int32).reshape(2 * N_EDGES),
        perm.astype(jnp.int32))
    return out_x, out_e.reshape(2, N_EDGES).astype(edge_index.dtype)


# speedup vs baseline: 247.6131x; 1.2081x over previous
import functools

import jax
import jax.numpy as jnp
from jax import lax
from jax.experimental import pallas as pl
from jax.experimental.pallas import tpu as pltpu
from jax.experimental.pallas import tpu_sc as plsc

N_NODES = 10000
D_FEAT = 128
N_EDGES = 320000

NC = 2
NS = 16
W = NC * NS
L = 16

ROWS_PW = 320
NRCH = ROWS_PW // L
EC = 9984
TAIL = N_EDGES - W * EC
TSTART = W * EC
SCAT_IT = N_NODES // L
GATH_IT = EC // L


def _sc_body(x_hbm, edge_hbm, perm_hbm, out_x, out_e,
             perm_v, table_v, eidx_v, eout_v, tidx_v, tout_v, rows_v,
             sem_p, sem_e, sem_g, sem_t):
    cid = lax.axis_index("c")
    sid = lax.axis_index("s")
    wid = sid * NC + cid
    ec = pl.multiple_of(wid * EC, 128)

    perm_cp = pltpu.async_copy(perm_hbm, perm_v, sem_p)
    edge_cp = pltpu.async_copy(edge_hbm.at[:, pl.ds(ec, EC)], eidx_v, sem_e)

    @pl.when(wid == 0)
    def _():
        pltpu.async_copy(edge_hbm.at[:, pl.ds(TSTART, TAIL)], tidx_v, sem_t)
    perm_cp.wait()
    row_cps = []
    for j in range(NRCH):
        base = jnp.minimum(wid * ROWS_PW + j * L, N_NODES - L)
        base = pl.multiple_of(base, 8)
        row_cps.append(pltpu.async_copy(
            x_hbm.at[perm_v.at[pl.ds(base, L)]], rows_v.at[j], sem_g))

    @plsc.parallel_loop(0, SCAT_IT * L, step=L, unroll=8)
    def _scat(sbase):
        sbase = pl.multiple_of(sbase, L)
        idxs = perm_v[pl.ds(sbase, L)]
        vals = lax.iota(jnp.int32, L) + sbase
        plsc.store_scatter(table_v, [idxs], vals)

    edge_cp.wait()

    @plsc.parallel_loop(0, GATH_IT * L, step=L, unroll=8)
    def _gath(gbase):
        gbase = pl.multiple_of(gbase, L)
        for r in range(2):
            idxs = eidx_v[r, pl.ds(gbase, L)]
            eout_v[r, pl.ds(gbase, L)] = plsc.load_gather(table_v, [idxs])

    pltpu.sync_copy(eout_v, out_e.at[:, pl.ds(ec, EC)])

    @pl.when(wid == 0)
    def _():
        pltpu.make_async_copy(
            edge_hbm.at[:, pl.ds(TSTART, TAIL)], tidx_v, sem_t).wait()

        @plsc.parallel_loop(0, TAIL, step=L, unroll=8)
        def _tgath(gbase):
            gbase = pl.multiple_of(gbase, L)
            for r in range(2):
                idxs = tidx_v[r, pl.ds(gbase, L)]
                tout_v[r, pl.ds(gbase, L)] = plsc.load_gather(table_v, [idxs])

        pltpu.sync_copy(tout_v, out_e.at[:, pl.ds(TSTART, TAIL)])

    for j in range(NRCH):
        row_cps[j].wait()
        base = pl.multiple_of(wid * ROWS_PW + j * L, L)

        @pl.when(base < N_NODES)
        def _():
            pltpu.sync_copy(rows_v.at[j], out_x.at[pl.ds(base, L)])


_sc_call = functools.partial(
    pl.kernel,
    out_type=[
        jax.ShapeDtypeStruct((N_NODES, D_FEAT), jnp.float32),
        jax.ShapeDtypeStruct((2, N_EDGES), jnp.int32),
    ],
    mesh=plsc.VectorSubcoreMesh(core_axis_name="c", subcore_axis_name="s"),
    compiler_params=pltpu.CompilerParams(needs_layout_passes=False),
    scratch_types=[
        pltpu.VMEM((N_NODES,), jnp.int32),
        pltpu.VMEM((N_NODES,), jnp.int32),
        pltpu.VMEM((2, EC), jnp.int32),
        pltpu.VMEM((2, EC), jnp.int32),
        pltpu.VMEM((2, TAIL), jnp.int32),
        pltpu.VMEM((2, TAIL), jnp.int32),
        pltpu.VMEM((NRCH, L, D_FEAT), jnp.float32),
        pltpu.SemaphoreType.DMA,
        pltpu.SemaphoreType.DMA,
        pltpu.SemaphoreType.DMA,
        pltpu.SemaphoreType.DMA,
    ],
)(_sc_body)


def kernel(x, edge_index, perm):
    out_x, out_e = _sc_call(
        x, edge_index.astype(jnp.int32), perm.astype(jnp.int32))
    return out_x, out_e.astype(edge_index.dtype)


# speedup vs baseline: 249.7101x; 1.0085x over previous
import functools

import jax
import jax.numpy as jnp
from jax import lax
from jax.experimental import pallas as pl
from jax.experimental.pallas import tpu as pltpu
from jax.experimental.pallas import tpu_sc as plsc

N_NODES = 10000
D_FEAT = 128
N_EDGES = 320000

NC = 2
NS = 16
W = NC * NS
L = 16

ROWS_PW = 320
NRCH = ROWS_PW // L
EC = 9984
TAIL = N_EDGES - W * EC
TSTART = W * EC
SCAT_IT = N_NODES // L
GATH_IT = EC // L


HALF = (GATH_IT // 2) * L


def _sc_body(x_hbm, edge_hbm, perm_hbm, out_x, out_e,
             perm_v, table_v, eidx_v, eout_v, tidx_v, tout_v, rows_v,
             sem_p, sem_e, sem_g, sem_t, sem_o):
    cid = lax.axis_index("c")
    sid = lax.axis_index("s")
    wid = sid * NC + cid
    ec = pl.multiple_of(wid * EC, 128)

    perm_cp = pltpu.async_copy(perm_hbm, perm_v, sem_p)
    edge_cp = pltpu.async_copy(edge_hbm.at[:, pl.ds(ec, EC)], eidx_v, sem_e)

    @pl.when(wid == 1)
    def _():
        pltpu.async_copy(edge_hbm.at[:, pl.ds(TSTART, TAIL)], tidx_v, sem_t)
    perm_cp.wait()
    row_cps = []
    for j in range(NRCH):
        base = jnp.minimum(wid * ROWS_PW + j * L, N_NODES - L)
        base = pl.multiple_of(base, 8)
        row_cps.append(pltpu.async_copy(
            x_hbm.at[perm_v.at[pl.ds(base, L)]], rows_v.at[j], sem_g))

    @plsc.parallel_loop(0, SCAT_IT * L, step=L, unroll=8)
    def _scat(sbase):
        sbase = pl.multiple_of(sbase, L)
        idxs = perm_v[pl.ds(sbase, L)]
        vals = lax.iota(jnp.int32, L) + sbase
        plsc.store_scatter(table_v, [idxs], vals)

    for j in range(NRCH):
        row_cps[j].wait()
        base = pl.multiple_of(wid * ROWS_PW + j * L, L)

        @pl.when(base < N_NODES)
        def _():
            pltpu.async_copy(rows_v.at[j], out_x.at[pl.ds(base, L)], sem_o)

    edge_cp.wait()

    for h in range(2):
        hoff = h * HALF

        @plsc.parallel_loop(hoff, hoff + HALF, step=L, unroll=8)
        def _gath(gbase):
            gbase = pl.multiple_of(gbase, L)
            for r in range(2):
                idxs = eidx_v[r, pl.ds(gbase, L)]
                eout_v[r, pl.ds(gbase, L)] = plsc.load_gather(table_v, [idxs])

        pltpu.async_copy(
            eout_v.at[:, pl.ds(hoff, HALF)],
            out_e.at[:, pl.ds(ec + hoff, HALF)], sem_o)

    @pl.when(wid == 1)
    def _():
        pltpu.make_async_copy(
            edge_hbm.at[:, pl.ds(TSTART, TAIL)], tidx_v, sem_t).wait()

        @plsc.parallel_loop(0, TAIL, step=L, unroll=8)
        def _tgath(gbase):
            gbase = pl.multiple_of(gbase, L)
            for r in range(2):
                idxs = tidx_v[r, pl.ds(gbase, L)]
                tout_v[r, pl.ds(gbase, L)] = plsc.load_gather(table_v, [idxs])

        pltpu.async_copy(tout_v, out_e.at[:, pl.ds(TSTART, TAIL)], sem_o)

    for j in range(NRCH):
        base = pl.multiple_of(wid * ROWS_PW + j * L, L)

        @pl.when(base < N_NODES)
        def _():
            pltpu.make_async_copy(
                rows_v.at[j], out_x.at[pl.ds(base, L)], sem_o).wait()

    for h in range(2):
        hoff = h * HALF
        pltpu.make_async_copy(
            eout_v.at[:, pl.ds(hoff, HALF)],
            out_e.at[:, pl.ds(ec + hoff, HALF)], sem_o).wait()

    @pl.when(wid == 1)
    def _():
        pltpu.make_async_copy(
            tout_v, out_e.at[:, pl.ds(TSTART, TAIL)], sem_o).wait()


_sc_call = functools.partial(
    pl.kernel,
    out_type=[
        jax.ShapeDtypeStruct((N_NODES, D_FEAT), jnp.float32),
        jax.ShapeDtypeStruct((2, N_EDGES), jnp.int32),
    ],
    mesh=plsc.VectorSubcoreMesh(core_axis_name="c", subcore_axis_name="s"),
    compiler_params=pltpu.CompilerParams(needs_layout_passes=False),
    scratch_types=[
        pltpu.VMEM((N_NODES,), jnp.int32),
        pltpu.VMEM((N_NODES,), jnp.int32),
        pltpu.VMEM((2, EC), jnp.int32),
        pltpu.VMEM((2, EC), jnp.int32),
        pltpu.VMEM((2, TAIL), jnp.int32),
        pltpu.VMEM((2, TAIL), jnp.int32),
        pltpu.VMEM((NRCH, L, D_FEAT), jnp.float32),
        pltpu.SemaphoreType.DMA,
        pltpu.SemaphoreType.DMA,
        pltpu.SemaphoreType.DMA,
        pltpu.SemaphoreType.DMA,
        pltpu.SemaphoreType.DMA,
    ],
)(_sc_body)


def kernel(x, edge_index, perm):
    out_x, out_e = _sc_call(
        x, edge_index.astype(jnp.int32), perm.astype(jnp.int32))
    return out_x, out_e.astype(edge_index.dtype)


# speedup vs baseline: 249.9984x; 1.0012x over previous
import functools

import jax
import jax.numpy as jnp
from jax import lax
from jax.experimental import pallas as pl
from jax.experimental.pallas import tpu as pltpu
from jax.experimental.pallas import tpu_sc as plsc

N_NODES = 10000
D_FEAT = 128
N_EDGES = 320000

NC = 2
NS = 16
W = NC * NS
L = 16

ROWS_PW = 320
RCH = 16
NRCH = ROWS_PW // RCH
EC = 9984
TAIL = N_EDGES - W * EC
TSTART = W * EC
SCAT_IT = N_NODES // L
GATH_IT = EC // L


HALF = (GATH_IT // 2) * L


def _sc_body(x_hbm, edge_hbm, perm_hbm, out_x, out_e,
             perm_v, table_v, eidx_v, eout_v, tidx_v, tout_v, rows_v,
             sem_p, sem_e, sem_g, sem_t, sem_o):
    cid = lax.axis_index("c")
    sid = lax.axis_index("s")
    wid = sid * NC + cid
    ec = pl.multiple_of(wid * EC, 128)

    perm_cp = pltpu.async_copy(perm_hbm, perm_v, sem_p)
    edge_cp = pltpu.async_copy(edge_hbm.at[:, pl.ds(ec, EC)], eidx_v, sem_e)

    @pl.when(wid == 1)
    def _():
        pltpu.async_copy(edge_hbm.at[:, pl.ds(TSTART, TAIL)], tidx_v, sem_t)
    perm_cp.wait()
    row_cps = []
    for j in range(NRCH):
        base = jnp.minimum(wid * ROWS_PW + j * RCH, N_NODES - RCH)
        base = pl.multiple_of(base, 8)
        row_cps.append(pltpu.async_copy(
            x_hbm.at[perm_v.at[pl.ds(base, RCH)]], rows_v.at[j], sem_g))

    @plsc.parallel_loop(0, SCAT_IT * L, step=L, unroll=8)
    def _scat(sbase):
        sbase = pl.multiple_of(sbase, L)
        idxs = perm_v[pl.ds(sbase, L)]
        vals = lax.iota(jnp.int32, L) + sbase
        plsc.store_scatter(table_v, [idxs], vals)

    for j in range(NRCH):
        row_cps[j].wait()
        base = pl.multiple_of(wid * ROWS_PW + j * RCH, 8)

        @pl.when(base < N_NODES)
        def _():
            pltpu.async_copy(rows_v.at[j], out_x.at[pl.ds(base, RCH)], sem_o)

    edge_cp.wait()

    for h in range(2):
        hoff = h * HALF

        @plsc.parallel_loop(hoff, hoff + HALF, step=L, unroll=8)
        def _gath(gbase):
            gbase = pl.multiple_of(gbase, L)
            for r in range(2):
                idxs = eidx_v[r, pl.ds(gbase, L)]
                eout_v[r, pl.ds(gbase, L)] = plsc.load_gather(table_v, [idxs])

        pltpu.async_copy(
            eout_v.at[:, pl.ds(hoff, HALF)],
            out_e.at[:, pl.ds(ec + hoff, HALF)], sem_o)

    @pl.when(wid == 1)
    def _():
        pltpu.make_async_copy(
            edge_hbm.at[:, pl.ds(TSTART, TAIL)], tidx_v, sem_t).wait()

        @plsc.parallel_loop(0, TAIL, step=L, unroll=8)
        def _tgath(gbase):
            gbase = pl.multiple_of(gbase, L)
            for r in range(2):
                idxs = tidx_v[r, pl.ds(gbase, L)]
                tout_v[r, pl.ds(gbase, L)] = plsc.load_gather(table_v, [idxs])

        pltpu.async_copy(tout_v, out_e.at[:, pl.ds(TSTART, TAIL)], sem_o)

    for j in range(NRCH):
        base = pl.multiple_of(wid * ROWS_PW + j * RCH, 8)

        @pl.when(base < N_NODES)
        def _():
            pltpu.make_async_copy(
                rows_v.at[j], out_x.at[pl.ds(base, RCH)], sem_o).wait()

    for h in range(2):
        hoff = h * HALF
        pltpu.make_async_copy(
            eout_v.at[:, pl.ds(hoff, HALF)],
            out_e.at[:, pl.ds(ec + hoff, HALF)], sem_o).wait()

    @pl.when(wid == 1)
    def _():
        pltpu.make_async_copy(
            tout_v, out_e.at[:, pl.ds(TSTART, TAIL)], sem_o).wait()


_sc_call = functools.partial(
    pl.kernel,
    out_type=[
        jax.ShapeDtypeStruct((N_NODES, D_FEAT), jnp.float32),
        jax.ShapeDtypeStruct((2, N_EDGES), jnp.int32),
    ],
    mesh=plsc.VectorSubcoreMesh(core_axis_name="c", subcore_axis_name="s"),
    compiler_params=pltpu.CompilerParams(needs_layout_passes=False),
    scratch_types=[
        pltpu.VMEM((N_NODES,), jnp.int32),
        pltpu.VMEM((N_NODES,), jnp.int32),
        pltpu.VMEM((2, EC), jnp.int32),
        pltpu.VMEM((2, EC), jnp.int32),
        pltpu.VMEM((2, TAIL), jnp.int32),
        pltpu.VMEM((2, TAIL), jnp.int32),
        pltpu.VMEM((NRCH, RCH, D_FEAT), jnp.float32),
        pltpu.SemaphoreType.DMA,
        pltpu.SemaphoreType.DMA,
        pltpu.SemaphoreType.DMA,
        pltpu.SemaphoreType.DMA,
        pltpu.SemaphoreType.DMA,
    ],
)(_sc_body)


def kernel(x, edge_index, perm):
    out_x, out_e = _sc_call(
        x, edge_index.astype(jnp.int32), perm.astype(jnp.int32))
    return out_x, out_e.astype(edge_index.dtype)
